# Initial kernel scaffold; baseline (speedup 1.0000x reference)
import jax
import jax.numpy as jnp
from jax.experimental import pallas as pl


def kernel(x_0, adjacency_0, att_parameter):
    raise NotImplementedError("write your pallas kernel here")


# speedup vs baseline: 4.9596x; 4.9596x over previous
import functools

import jax
import jax.numpy as jnp
from jax import lax
from jax.experimental import pallas as pl
from jax.experimental.pallas import tpu as pltpu
from jax.experimental.pallas import tpu_sc as plsc

_N = 10000
_C = 128
_H = 4
_E = 320000

_NC = 2
_NS = 16
_NW = _NC * _NS
_EPW = _E // _NW
_CHUNK = 2000
_NCHUNK = _EPW // _CHUNK
_ITERS = _CHUNK // 4


def _mm_body(x_ref, w_ref, o_ref):
    o_ref[...] = jnp.dot(x_ref[...], w_ref[...],
                         preferred_element_type=jnp.float32)


def _node_projection(x, w):
    return pl.pallas_call(
        _mm_body,
        out_shape=jax.ShapeDtypeStruct((_N, 2 * _H), jnp.float32),
    )(x, w)


_mesh = plsc.VectorSubcoreMesh(core_axis_name="c", subcore_axis_name="s",
                               num_cores=_NC, num_subcores=_NS)


@functools.partial(
    pl.kernel,
    out_type=jax.ShapeDtypeStruct((_E * _H,), jnp.float32),
    mesh=_mesh,
    compiler_params=pltpu.CompilerParams(needs_layout_passes=False),
    scratch_types=[
        pltpu.VMEM((_N * 2 * _H,), jnp.float32),
        pltpu.VMEM((_CHUNK,), jnp.int32),
        pltpu.VMEM((_CHUNK,), jnp.int32),
        pltpu.VMEM((_CHUNK * _H,), jnp.float32),
    ],
)
def _sc_edges(tbl_hbm, src_hbm, tgt_hbm, out_hbm, tbl_v, src_v, tgt_v, out_v):
    wid = lax.axis_index("s") * _NC + lax.axis_index("c")
    pltpu.sync_copy(tbl_hbm, tbl_v)

    iota = lax.iota(jnp.int32, 16)
    hpat = lax.bitwise_and(iota, 3)
    erel = lax.shift_right_logical(iota, 2)

    base_e = wid * _EPW
    for c in range(_NCHUNK):
        ebase = base_e + c * _CHUNK
        pltpu.sync_copy(src_hbm.at[pl.ds(ebase, _CHUNK)], src_v)
        pltpu.sync_copy(tgt_hbm.at[pl.ds(ebase, _CHUNK)], tgt_v)

        def body(i, carry):
            rep = i * 4 + erel
            s = plsc.load_gather(src_v, [rep])
            t = plsc.load_gather(tgt_v, [rep])
            u = plsc.load_gather(tbl_v, [s * (2 * _H) + hpat])
            v = plsc.load_gather(tbl_v, [t * (2 * _H) + (hpat + _H)])
            out_v[pl.ds(i * 16, 16)] = jnp.maximum(u + v, 0.0)
            return carry

        lax.fori_loop(0, _ITERS, body, 0)
        pltpu.sync_copy(out_v, out_hbm.at[pl.ds(ebase * _H, _CHUNK * _H)])


def kernel(x_0, adjacency_0, att_parameter):
    adj = adjacency_0.astype(jnp.int32)
    w = jnp.concatenate([att_parameter[:_C], att_parameter[_C:]], axis=1)
    p = _node_projection(x_0, w)
    out_flat = _sc_edges(p.reshape(-1), adj[0], adj[1])
    return out_flat.reshape(_E, _H)


# speedup vs baseline: 34.8665x; 7.0301x over previous
import functools

import jax
import jax.numpy as jnp
from jax import lax
from jax.experimental import pallas as pl
from jax.experimental.pallas import tpu as pltpu
from jax.experimental.pallas import tpu_sc as plsc

_N = 10000
_C = 128
_H = 4
_E = 320000

_NC = 2
_NS = 16
_NW = _NC * _NS

_BLK = 128
_NBLK = _E // _BLK
_CB = 10
_NTASK = _NBLK // _CB
_EPT = _CB * _BLK
_OPT = _EPT * _H
_KMAX = (_NTASK + _NW - 1) // _NW
_MITER = _EPT // 16


def _mm_body(x_ref, w_ref, o_ref):
    o_ref[...] = jnp.dot(x_ref[...], w_ref[...],
                         preferred_element_type=jnp.float32)


def _node_projection(x, w):
    return pl.pallas_call(
        _mm_body,
        out_shape=jax.ShapeDtypeStruct((_N, 2 * _H), jnp.float32),
    )(x, w)


_mesh = plsc.VectorSubcoreMesh(core_axis_name="c", subcore_axis_name="s",
                               num_cores=_NC, num_subcores=_NS)


@functools.partial(
    pl.kernel,
    out_type=jax.ShapeDtypeStruct((_E * _H,), jnp.float32),
    mesh=_mesh,
    compiler_params=pltpu.CompilerParams(needs_layout_passes=False),
    scratch_types=[
        pltpu.VMEM((_N * 2 * _H,), jnp.float32),
        pltpu.VMEM((_EPT,), jnp.int32),
        pltpu.VMEM((_EPT,), jnp.int32),
        pltpu.VMEM((_EPT,), jnp.int32),
        pltpu.VMEM((_EPT,), jnp.int32),
        pltpu.VMEM((_OPT,), jnp.float32),
        pltpu.VMEM((_OPT,), jnp.float32),
        pltpu.SemaphoreType.DMA,
        pltpu.SemaphoreType.DMA,
        pltpu.SemaphoreType.DMA,
        pltpu.SemaphoreType.DMA,
    ],
)
def _sc_edges(tbl_hbm, adj_hbm, out_hbm, tbl_v,
              src_v0, src_v1, tgt_v0, tgt_v1, out_v0, out_v1,
              sin0, sin1, sout0, sout1):
    wid = lax.axis_index("s") * _NC + lax.axis_index("c")
    pltpu.sync_copy(tbl_hbm, tbl_v)

    srcs = (src_v0, src_v1)
    tgts = (tgt_v0, tgt_v1)
    outs = (out_v0, out_v1)
    sins = (sin0, sin1)
    souts = (sout0, sout1)

    def task_id(k):
        return wid + _NW * k

    def task_valid(k):
        return task_id(k) < _NTASK

    def start_in(k):
        b = k % 2
        off = task_id(k) * _EPT
        pltpu.async_copy(adj_hbm.at[0, pl.ds(off, _EPT)], srcs[b], sins[b])
        pltpu.async_copy(adj_hbm.at[1, pl.ds(off, _EPT)], tgts[b], sins[b])

    def wait_in(k):
        b = k % 2
        off = task_id(k) * _EPT
        pltpu.make_async_copy(adj_hbm.at[0, pl.ds(off, _EPT)], srcs[b],
                              sins[b]).wait()
        pltpu.make_async_copy(adj_hbm.at[1, pl.ds(off, _EPT)], tgts[b],
                              sins[b]).wait()

    def start_out(k):
        b = k % 2
        off = task_id(k) * _OPT
        pltpu.async_copy(outs[b], out_hbm.at[pl.ds(off, _OPT)], souts[b])

    def wait_out(k):
        b = k % 2
        off = task_id(k) * _OPT
        pltpu.make_async_copy(outs[b], out_hbm.at[pl.ds(off, _OPT)],
                              souts[b]).wait()

    def compute(k):
        b = k % 2
        src_v, tgt_v, out_v = srcs[b], tgts[b], outs[b]

        @plsc.parallel_loop(0, _MITER, unroll=4)
        def body(m):
            s = src_v[pl.ds(m * 16, 16)]
            t = tgt_v[pl.ds(m * 16, 16)]
            s8 = s * (2 * _H)
            t8 = t * (2 * _H) + _H
            obase = (m // 8) * (_H * _BLK) + (m % 8) * 16
            for h in range(_H):
                u = plsc.load_gather(tbl_v, [s8 + h])
                v = plsc.load_gather(tbl_v, [t8 + h])
                out_v[pl.ds(obase + h * _BLK, 16)] = jnp.maximum(u + v, 0.0)

    start_in(0)
    for k in range(_KMAX):
        if k + 1 < _KMAX:
            if k + 1 == _KMAX - 1:
                @pl.when(task_valid(k + 1))
                def _():
                    start_in(k + 1)
            else:
                start_in(k + 1)
        if k == _KMAX - 1:
            @pl.when(task_valid(k))
            def _():
                wait_in(k)
                wait_out(k - 2)
                compute(k)
                start_out(k)
        else:
            wait_in(k)
            if k >= 2:
                wait_out(k - 2)
            compute(k)
            start_out(k)
    wait_out(_KMAX - 2)

    @pl.when(task_valid(_KMAX - 1))
    def _():
        wait_out(_KMAX - 1)


def kernel(x_0, adjacency_0, att_parameter):
    adj = adjacency_0.astype(jnp.int32)
    w = jnp.concatenate([att_parameter[:_C], att_parameter[_C:]], axis=1)
    p = _node_projection(x_0, w)
    out_flat = _sc_edges(p.reshape(-1), adj)
    out = out_flat.reshape(_NBLK, _H, _BLK).transpose(0, 2, 1).reshape(_E, _H)
    return out


# speedup vs baseline: 41.6820x; 1.1955x over previous
import functools

import jax
import jax.numpy as jnp
from jax import lax
from jax.experimental import pallas as pl
from jax.experimental.pallas import tpu as pltpu
from jax.experimental.pallas import tpu_sc as plsc

_N = 10000
_C = 128
_H = 4
_E = 320000

_NC = 2
_NS = 16
_NW = _NC * _NS

_BLK = 128
_NBLK = _E // _BLK
_CB = 10
_NTASK = _NBLK // _CB
_EPT = _CB * _BLK
_OPT = _EPT * _H
_KMAX = (_NTASK + _NW - 1) // _NW
_MITER = _EPT // 16


def _mm_body(w_ref, x_ref, o_ref):
    o_ref[...] = lax.dot_general(
        w_ref[...], x_ref[...], (((0,), (1,)), ((), ())),
        preferred_element_type=jnp.float32)


def _node_projection(x, w):
    return pl.pallas_call(
        _mm_body,
        out_shape=jax.ShapeDtypeStruct((2 * _H, _N), jnp.float32),
    )(w, x)


_mesh = plsc.VectorSubcoreMesh(core_axis_name="c", subcore_axis_name="s",
                               num_cores=_NC, num_subcores=_NS)


@functools.partial(
    pl.kernel,
    out_type=jax.ShapeDtypeStruct((_E * _H,), jnp.float32),
    mesh=_mesh,
    compiler_params=pltpu.CompilerParams(needs_layout_passes=False),
    scratch_types=[
        pltpu.VMEM((_N * 2 * _H,), jnp.float32),
        pltpu.VMEM((_EPT,), jnp.int32),
        pltpu.VMEM((_EPT,), jnp.int32),
        pltpu.VMEM((_EPT,), jnp.int32),
        pltpu.VMEM((_EPT,), jnp.int32),
        pltpu.VMEM((_OPT,), jnp.float32),
        pltpu.VMEM((_OPT,), jnp.float32),
        pltpu.SemaphoreType.DMA,
        pltpu.SemaphoreType.DMA,
        pltpu.SemaphoreType.DMA,
        pltpu.SemaphoreType.DMA,
    ],
)
def _sc_edges(tbl_hbm, adj_hbm, out_hbm, tbl_v,
              src_v0, src_v1, tgt_v0, tgt_v1, out_v0, out_v1,
              sin0, sin1, sout0, sout1):
    wid = lax.axis_index("s") * _NC + lax.axis_index("c")

    srcs = (src_v0, src_v1)
    tgts = (tgt_v0, tgt_v1)
    outs = (out_v0, out_v1)
    sins = (sin0, sin1)
    souts = (sout0, sout1)

    def task_id(k):
        return wid + _NW * k

    def task_valid(k):
        return task_id(k) < _NTASK

    def start_in(k):
        b = k % 2
        off = task_id(k) * _EPT
        pltpu.async_copy(adj_hbm.at[0, pl.ds(off, _EPT)], srcs[b], sins[b])
        pltpu.async_copy(adj_hbm.at[1, pl.ds(off, _EPT)], tgts[b], sins[b])

    def wait_in(k):
        b = k % 2
        off = task_id(k) * _EPT
        pltpu.make_async_copy(adj_hbm.at[0, pl.ds(off, _EPT)], srcs[b],
                              sins[b]).wait()
        pltpu.make_async_copy(adj_hbm.at[1, pl.ds(off, _EPT)], tgts[b],
                              sins[b]).wait()

    def start_out(k):
        b = k % 2
        off = task_id(k) * _OPT
        pltpu.async_copy(outs[b], out_hbm.at[pl.ds(off, _OPT)], souts[b])

    def wait_out(k):
        b = k % 2
        off = task_id(k) * _OPT
        pltpu.make_async_copy(outs[b], out_hbm.at[pl.ds(off, _OPT)],
                              souts[b]).wait()

    def compute(k):
        b = k % 2
        src_v, tgt_v, out_v = srcs[b], tgts[b], outs[b]

        @plsc.parallel_loop(0, _MITER, unroll=4)
        def body(m):
            s = src_v[pl.ds(m * 16, 16)]
            t = tgt_v[pl.ds(m * 16, 16)]
            obase = (m // 8) * (_H * _BLK) + (m % 8) * 16
            for h in range(_H):
                u = plsc.load_gather(tbl_v, [s + (h * _N)])
                v = plsc.load_gather(tbl_v, [t + ((_H + h) * _N)])
                out_v[pl.ds(obase + h * _BLK, 16)] = jnp.maximum(u + v, 0.0)

    start_in(0)
    pltpu.sync_copy(tbl_hbm, tbl_v)
    for k in range(_KMAX):
        if k + 1 < _KMAX:
            if k + 1 == _KMAX - 1:
                @pl.when(task_valid(k + 1))
                def _():
                    start_in(k + 1)
            else:
                start_in(k + 1)
        if k == _KMAX - 1:
            @pl.when(task_valid(k))
            def _():
                wait_in(k)
                wait_out(k - 2)
                compute(k)
                start_out(k)
        else:
            wait_in(k)
            if k >= 2:
                wait_out(k - 2)
            compute(k)
            start_out(k)
    wait_out(_KMAX - 2)

    @pl.when(task_valid(_KMAX - 1))
    def _():
        wait_out(_KMAX - 1)


def kernel(x_0, adjacency_0, att_parameter):
    adj = adjacency_0.astype(jnp.int32)
    w = jnp.concatenate([att_parameter[:_C], att_parameter[_C:]], axis=1)
    p = _node_projection(x_0, w)
    out_flat = _sc_edges(p.reshape(-1), adj)
    out = out_flat.reshape(_NBLK, _H, _BLK).transpose(0, 2, 1).reshape(_E, _H)
    return out
